# baseline (speedup 1.0000x reference)
import functools

import jax
import jax.numpy as jnp
from jax.experimental import pallas as pl
from jax.experimental.pallas import tpu as pltpu

_E = 8
_C = 192
_HID = 192
_B = 16
_S = 16
_P = _S * _S
_PAD = 128

_TAPS = [(k, 16 * (k // 3 - 1) + (k % 3 - 1), k % 3 - 1) for k in range(9)]


def _gate_kernel(meta_ref, w1_ref, b1_ref, w2_ref, b2_ref, top1_ref, ord_ref,
                 bal_ref):
    meta = meta_ref[:]
    h = jax.lax.dot_general(meta, w1_ref[:], (((1,), (1,)), ((), ())),
                            preferred_element_type=jnp.float32)
    h = jnp.maximum(h + b1_ref[:], 0.0)
    logits = jax.lax.dot_general(h, w2_ref[:], (((1,), (1,)), ((), ())),
                                 preferred_element_type=jnp.float32)
    logits = logits + b2_ref[:]
    mx = jnp.max(logits, axis=1, keepdims=True)
    ex = jnp.exp(logits - mx)
    probs = ex / jnp.sum(ex, axis=1, keepdims=True)
    lane = jax.lax.broadcasted_iota(jnp.int32, (_B, _E), 1)
    is_max = logits == mx
    top1 = jnp.min(jnp.where(is_max, lane, _E), axis=1, keepdims=True)
    top1_ref[:] = top1
    ecol = jnp.transpose(top1)
    bcol = jax.lax.broadcasted_iota(jnp.int32, (1, _B), 1)
    brow = jax.lax.broadcasted_iota(jnp.int32, (_B, 1), 0)
    less = (ecol < top1).astype(jnp.int32)
    eqpr = ((ecol == top1) & (bcol < brow)).astype(jnp.int32)
    rank = jnp.sum(less + eqpr, axis=1, keepdims=True)
    sel = (jnp.transpose(rank) == brow).astype(jnp.int32)
    ord_ref[:] = jnp.sum(sel * bcol, axis=1, keepdims=True)
    imp = jnp.sum(probs, axis=0, keepdims=True)
    imp = imp / (jnp.sum(imp, axis=1, keepdims=True) + 1e-8)
    mean = jnp.sum(imp, axis=1, keepdims=True) / _E
    var = jnp.sum((imp - mean) ** 2, axis=1, keepdims=True) / (_E - 1)
    bal_ref[:, :] = jnp.sqrt(var)


def _moe_kernel(top1_ref, ord_ref, x_ref, w1_ref, w2_ref,
                bn1_g_ref, bn1_b_ref, bn2_g_ref, bn2_b_ref,
                out_ref, h1_ref, h2_ref, hpad_ref,
                sc1_ref, sh1_ref, sc2_ref, sh2_ref, wbuf_ref, wbf_ref, sem):
    f32 = jnp.float32
    row = jax.lax.broadcasted_iota(jnp.int32, (_P, 1), 0)
    mask_m = (row % _S != 0).astype(f32)
    mask_p = (row % _S != _S - 1).astype(f32)

    def fetch(w_ref, e, slot):
        pltpu.make_async_copy(
            w_ref.at[e], wbuf_ref.at[slot], sem.at[slot]).start()

    def ready(w_ref, e, slot):
        pltpu.make_async_copy(
            w_ref.at[e], wbuf_ref.at[slot], sem.at[slot]).wait()
        wbf_ref[slot] = wbuf_ref[slot].astype(jnp.bfloat16)

    def next_change(i, e):
        b_nxt = ord_ref[jnp.minimum(i + 1, _B - 1), 0]
        e_nxt = top1_ref[b_nxt, 0]
        return e_nxt, jnp.logical_and(i + 1 < _B, e_nxt != e)

    def conv(slot):
        by_dx = {-1: None, 0: None, 1: None}
        for k, off, dx in _TAPS:
            xs = hpad_ref[_PAD + off:_PAD + off + _P, :]
            t = jax.lax.dot_general(xs, wbf_ref[slot, k],
                                    (((1,), (1,)), ((), ())),
                                    preferred_element_type=f32)
            by_dx[dx] = t if by_dx[dx] is None else by_dx[dx] + t
        return by_dx[-1] * mask_m + by_dx[0] + by_dx[1] * mask_p

    def affine(ssum, ssq, cnt, g_ref, b_ref, sc_ref, sh_ref):
        for e in range(_E):
            n = jnp.maximum(cnt[e:e + 1, :], 1.0) * _P
            m = ssum[e:e + 1, :] / n
            v = ssq[e:e + 1, :] / n - m * m
            sc = g_ref[e:e + 1, :] * jax.lax.rsqrt(v + 1e-5)
            sc_ref[e] = sc
            sh_ref[e] = b_ref[e:e + 1, :] - m * sc

    zstat = jnp.zeros((_E, _C), f32)
    zcnt = jnp.zeros((_E, 1), f32)
    hpad_ref[:, :] = jnp.zeros((_P + 2 * _PAD, _C), jnp.bfloat16)
    fetch(w1_ref, top1_ref[ord_ref[0, 0], 0], 0)
    ready(w1_ref, top1_ref[ord_ref[0, 0], 0], 0)

    def pass1(i, carry):
        ssum, ssq, cnt, slot = carry
        b = ord_ref[i, 0]
        e = top1_ref[b, 0]
        e_nxt, change = next_change(i, e)
        jax.lax.cond(change,
                     lambda: fetch(w1_ref, e_nxt, 1 - slot), lambda: None)
        hpad_ref[_PAD:_PAD + _P, :] = x_ref[b].astype(jnp.bfloat16)
        h = conv(slot)
        h1_ref[b] = h
        oh = (jax.lax.broadcasted_iota(jnp.int32, (_E, 1), 0) == e).astype(f32)
        ssum = ssum + oh * jnp.sum(h, axis=0, keepdims=True)
        ssq = ssq + oh * jnp.sum(h * h, axis=0, keepdims=True)
        cnt = cnt + oh
        jax.lax.cond(change,
                     lambda: ready(w1_ref, e_nxt, 1 - slot), lambda: None)
        slot = jnp.where(change, 1 - slot, slot)
        return ssum, ssq, cnt, slot

    ssum1, ssq1, cnt, _ = jax.lax.fori_loop(0, _B, pass1,
                                            (zstat, zstat, zcnt, 0))
    affine(ssum1, ssq1, cnt, bn1_g_ref, bn1_b_ref, sc1_ref, sh1_ref)

    fetch(w2_ref, top1_ref[ord_ref[0, 0], 0], 0)
    ready(w2_ref, top1_ref[ord_ref[0, 0], 0], 0)

    def pass2(i, carry):
        ssum, ssq, slot = carry
        b = ord_ref[i, 0]
        e = top1_ref[b, 0]
        e_nxt, change = next_change(i, e)
        jax.lax.cond(change,
                     lambda: fetch(w2_ref, e_nxt, 1 - slot), lambda: None)
        hn = jnp.maximum(h1_ref[b] * sc1_ref[e] + sh1_ref[e], 0.0)
        hpad_ref[_PAD:_PAD + _P, :] = hn.astype(jnp.bfloat16)
        h = conv(slot)
        h2_ref[b] = h
        oh = (jax.lax.broadcasted_iota(jnp.int32, (_E, 1), 0) == e).astype(f32)
        ssum = ssum + oh * jnp.sum(h, axis=0, keepdims=True)
        ssq = ssq + oh * jnp.sum(h * h, axis=0, keepdims=True)
        jax.lax.cond(change,
                     lambda: ready(w2_ref, e_nxt, 1 - slot), lambda: None)
        slot = jnp.where(change, 1 - slot, slot)
        return ssum, ssq, slot

    ssum2, ssq2, _ = jax.lax.fori_loop(0, _B, pass2, (zstat, zstat, 0))
    affine(ssum2, ssq2, cnt, bn2_g_ref, bn2_b_ref, sc2_ref, sh2_ref)

    def pass3(b, _):
        e = top1_ref[b, 0]
        out_ref[b] = jnp.maximum(h2_ref[b] * sc2_ref[e] + sh2_ref[e], 0.0)
        return 0

    jax.lax.fori_loop(0, _B, pass3, 0)


@functools.partial(jax.jit, static_argnames=("interpret",))
def kernel(moe_c4, meta, gate_w1, gate_b1, gate_w2, gate_b2, conv1_w, conv1_b,
           bn1_g, bn1_b, conv2_w, conv2_b, bn2_g, bn2_b, interpret=False):
    del conv1_b, conv2_b
    f32 = jnp.float32

    top1, order, bal = pl.pallas_call(
        _gate_kernel,
        out_shape=(jax.ShapeDtypeStruct((_B, 1), jnp.int32),
                   jax.ShapeDtypeStruct((_B, 1), jnp.int32),
                   jax.ShapeDtypeStruct((1, 1), f32)),
        interpret=interpret,
    )(meta, gate_w1, gate_b1[None, :], gate_w2, gate_b2[None, :])

    x_nat = moe_c4.transpose(0, 2, 3, 1).reshape(_B, _P, _C)
    w1p = conv1_w.transpose(0, 3, 4, 1, 2).reshape(_E, 9, _HID, _C)
    w2p = conv2_w.transpose(0, 3, 4, 1, 2).reshape(_E, 9, _C, _HID)

    out = pl.pallas_call(
        _moe_kernel,
        out_shape=jax.ShapeDtypeStruct((_B, _P, _C), f32),
        in_specs=[
            pl.BlockSpec(memory_space=pltpu.SMEM),
            pl.BlockSpec(memory_space=pltpu.SMEM),
            pl.BlockSpec(memory_space=pltpu.VMEM),
            pl.BlockSpec(memory_space=pltpu.MemorySpace.HBM),
            pl.BlockSpec(memory_space=pltpu.MemorySpace.HBM),
            pl.BlockSpec(memory_space=pltpu.VMEM),
            pl.BlockSpec(memory_space=pltpu.VMEM),
            pl.BlockSpec(memory_space=pltpu.VMEM),
            pl.BlockSpec(memory_space=pltpu.VMEM),
        ],
        scratch_shapes=[
            pltpu.VMEM((_B, _P, _HID), f32),
            pltpu.VMEM((_B, _P, _C), f32),
            pltpu.VMEM((_P + 2 * _PAD, _C), jnp.bfloat16),
            pltpu.VMEM((_E, 1, _HID), f32),
            pltpu.VMEM((_E, 1, _HID), f32),
            pltpu.VMEM((_E, 1, _C), f32),
            pltpu.VMEM((_E, 1, _C), f32),
            pltpu.VMEM((2, 9, _HID, _C), f32),
            pltpu.VMEM((2, 9, _HID, _C), jnp.bfloat16),
            pltpu.SemaphoreType.DMA((2,)),
        ],
        interpret=interpret,
    )(top1, order, x_nat, w1p, w2p, bn1_g, bn1_b, bn2_g, bn2_b)

    return out.reshape(_B, _S, _S, _C).transpose(0, 3, 1, 2), bal[0, 0]


# speedup vs baseline: 1.0007x; 1.0007x over previous
import functools

import jax
import jax.numpy as jnp
from jax.experimental import pallas as pl
from jax.experimental.pallas import tpu as pltpu

_E = 8
_C = 192
_HID = 192
_B = 16
_S = 16
_P = _S * _S
_PAD = 128

_TAPS = [(k, 16 * (k // 3 - 1) + (k % 3 - 1), k % 3 - 1) for k in range(9)]


def _gate_kernel(meta_ref, w1_ref, b1_ref, w2_ref, b2_ref, top1_ref, bal_ref):
    meta = meta_ref[:]
    h = jax.lax.dot_general(meta, w1_ref[:], (((1,), (1,)), ((), ())),
                            preferred_element_type=jnp.float32)
    h = jnp.maximum(h + b1_ref[:], 0.0)
    logits = jax.lax.dot_general(h, w2_ref[:], (((1,), (1,)), ((), ())),
                                 preferred_element_type=jnp.float32)
    logits = logits + b2_ref[:]
    mx = jnp.max(logits, axis=1, keepdims=True)
    ex = jnp.exp(logits - mx)
    probs = ex / jnp.sum(ex, axis=1, keepdims=True)
    lane = jax.lax.broadcasted_iota(jnp.int32, (_B, _E), 1)
    is_max = logits == mx
    top1 = jnp.min(jnp.where(is_max, lane, _E), axis=1, keepdims=True)
    top1_ref[:] = top1
    imp = jnp.sum(probs, axis=0, keepdims=True)
    imp = imp / (jnp.sum(imp, axis=1, keepdims=True) + 1e-8)
    mean = jnp.sum(imp, axis=1, keepdims=True) / _E
    var = jnp.sum((imp - mean) ** 2, axis=1, keepdims=True) / (_E - 1)
    bal_ref[:, :] = jnp.sqrt(var)


def _moe_kernel(top1_ref, x_ref, w1_ref, w2_ref,
                bn1_g_ref, bn1_b_ref, bn2_g_ref, bn2_b_ref,
                out_ref, h1_ref, h2_ref, xpad_ref, hpad_ref,
                sc1_ref, sh1_ref, sc2_ref, sh2_ref, wbuf_ref, wbf_ref, sem):
    f32 = jnp.float32
    bf16 = jnp.bfloat16
    row = jax.lax.broadcasted_iota(jnp.int32, (_P, 1), 0)
    mask_m = (row % _S != 0).astype(f32)
    mask_p = (row % _S != _S - 1).astype(f32)

    def fetch(w_ref, b, slot):
        e = top1_ref[b, 0]
        pltpu.make_async_copy(
            w_ref.at[e], wbuf_ref.at[slot], sem.at[slot]).start()

    def wait(w_ref, b, slot):
        e = top1_ref[b, 0]
        pltpu.make_async_copy(
            w_ref.at[e], wbuf_ref.at[slot], sem.at[slot]).wait()

    def conv(src, slot):
        by_dx = {-1: None, 0: None, 1: None}
        for k, off, dx in _TAPS:
            xs = src[_PAD + off:_PAD + off + _P, :]
            t = jax.lax.dot_general(xs, wbf_ref[slot, k],
                                    (((1,), (1,)), ((), ())),
                                    preferred_element_type=f32)
            by_dx[dx] = t if by_dx[dx] is None else by_dx[dx] + t
        return by_dx[-1] * mask_m + by_dx[0] + by_dx[1] * mask_p

    def affine(ssum, ssq, cnt, g_ref, b_ref, sc_ref, sh_ref):
        for e in range(_E):
            n = jnp.maximum(cnt[e:e + 1, :], 1.0) * _P
            m = ssum[e:e + 1, :] / n
            v = ssq[e:e + 1, :] / n - m * m
            sc = g_ref[e:e + 1, :] * jax.lax.rsqrt(v + 1e-5)
            sc_ref[e] = sc
            sh_ref[e] = b_ref[e:e + 1, :] - m * sc

    xpad_ref[:, :, :] = jnp.zeros((_B, _P + 2 * _PAD, _C), bf16)
    hpad_ref[:, :, :] = jnp.zeros((2, _P + 2 * _PAD, _C), bf16)

    def stage(b, _):
        xpad_ref[b, _PAD:_PAD + _P, :] = x_ref[b].astype(bf16)
        return 0

    jax.lax.fori_loop(0, _B, stage, 0)

    zstat = jnp.zeros((_E, _C), f32)
    zcnt = jnp.zeros((_E, 1), f32)
    fetch(w1_ref, 0, 0)

    def pass1(b, carry):
        ssum, ssq, cnt = carry
        e = top1_ref[b, 0]
        slot = jnp.bitwise_and(b, 1)
        jax.lax.cond(b + 1 < _B,
                     lambda: fetch(w1_ref, b + 1, 1 - slot), lambda: None)
        wait(w1_ref, b, slot)
        wbf_ref[slot] = wbuf_ref[slot].astype(bf16)
        h = conv(xpad_ref.at[b], slot)
        h1_ref[b] = h
        oh = (jax.lax.broadcasted_iota(jnp.int32, (_E, 1), 0) == e).astype(f32)
        ssum = ssum + oh * jnp.sum(h, axis=0, keepdims=True)
        ssq = ssq + oh * jnp.sum(h * h, axis=0, keepdims=True)
        cnt = cnt + oh
        return ssum, ssq, cnt

    ssum1, ssq1, cnt = jax.lax.fori_loop(0, _B, pass1, (zstat, zstat, zcnt))
    affine(ssum1, ssq1, cnt, bn1_g_ref, bn1_b_ref, sc1_ref, sh1_ref)

    fetch(w2_ref, 0, 0)

    def pass2(b, carry):
        ssum, ssq = carry
        e = top1_ref[b, 0]
        slot = jnp.bitwise_and(b, 1)
        jax.lax.cond(b + 1 < _B,
                     lambda: fetch(w2_ref, b + 1, 1 - slot), lambda: None)
        hn = jnp.maximum(h1_ref[b] * sc1_ref[e] + sh1_ref[e], 0.0)
        hpad_ref[slot, _PAD:_PAD + _P, :] = hn.astype(bf16)
        wait(w2_ref, b, slot)
        wbf_ref[slot] = wbuf_ref[slot].astype(bf16)
        h = conv(hpad_ref.at[slot], slot)
        h2_ref[b] = h
        oh = (jax.lax.broadcasted_iota(jnp.int32, (_E, 1), 0) == e).astype(f32)
        ssum = ssum + oh * jnp.sum(h, axis=0, keepdims=True)
        ssq = ssq + oh * jnp.sum(h * h, axis=0, keepdims=True)
        return ssum, ssq

    ssum2, ssq2 = jax.lax.fori_loop(0, _B, pass2, (zstat, zstat))
    affine(ssum2, ssq2, cnt, bn2_g_ref, bn2_b_ref, sc2_ref, sh2_ref)

    def pass3(b, _):
        e = top1_ref[b, 0]
        out_ref[b] = jnp.maximum(h2_ref[b] * sc2_ref[e] + sh2_ref[e], 0.0)
        return 0

    jax.lax.fori_loop(0, _B, pass3, 0)


@functools.partial(jax.jit, static_argnames=("interpret",))
def kernel(moe_c4, meta, gate_w1, gate_b1, gate_w2, gate_b2, conv1_w, conv1_b,
           bn1_g, bn1_b, conv2_w, conv2_b, bn2_g, bn2_b, interpret=False):
    del conv1_b, conv2_b
    f32 = jnp.float32

    top1, bal = pl.pallas_call(
        _gate_kernel,
        out_shape=(jax.ShapeDtypeStruct((_B, 1), jnp.int32),
                   jax.ShapeDtypeStruct((1, 1), f32)),
        interpret=interpret,
    )(meta, gate_w1, gate_b1[None, :], gate_w2, gate_b2[None, :])

    x_nat = moe_c4.transpose(0, 2, 3, 1).reshape(_B, _P, _C)
    w1p = conv1_w.transpose(0, 3, 4, 1, 2).reshape(_E, 9, _HID, _C)
    w2p = conv2_w.transpose(0, 3, 4, 1, 2).reshape(_E, 9, _C, _HID)

    out = pl.pallas_call(
        _moe_kernel,
        out_shape=jax.ShapeDtypeStruct((_B, _P, _C), f32),
        in_specs=[
            pl.BlockSpec(memory_space=pltpu.SMEM),
            pl.BlockSpec(memory_space=pltpu.VMEM),
            pl.BlockSpec(memory_space=pltpu.MemorySpace.HBM),
            pl.BlockSpec(memory_space=pltpu.MemorySpace.HBM),
            pl.BlockSpec(memory_space=pltpu.VMEM),
            pl.BlockSpec(memory_space=pltpu.VMEM),
            pl.BlockSpec(memory_space=pltpu.VMEM),
            pl.BlockSpec(memory_space=pltpu.VMEM),
        ],
        scratch_shapes=[
            pltpu.VMEM((_B, _P, _HID), f32),
            pltpu.VMEM((_B, _P, _C), f32),
            pltpu.VMEM((_B, _P + 2 * _PAD, _C), jnp.bfloat16),
            pltpu.VMEM((2, _P + 2 * _PAD, _C), jnp.bfloat16),
            pltpu.VMEM((_E, 1, _HID), f32),
            pltpu.VMEM((_E, 1, _HID), f32),
            pltpu.VMEM((_E, 1, _C), f32),
            pltpu.VMEM((_E, 1, _C), f32),
            pltpu.VMEM((2, 9, _HID, _C), f32),
            pltpu.VMEM((2, 9, _HID, _C), jnp.bfloat16),
            pltpu.SemaphoreType.DMA((2,)),
        ],
        interpret=interpret,
    )(top1, x_nat, w1p, w2p, bn1_g, bn1_b, bn2_g, bn2_b)

    return out.reshape(_B, _S, _S, _C).transpose(0, 3, 1, 2), bal[0, 0]


# speedup vs baseline: 1.2587x; 1.2578x over previous
import functools

import jax
import jax.numpy as jnp
from jax.experimental import pallas as pl
from jax.experimental.pallas import tpu as pltpu

_E = 8
_C = 192
_HID = 192
_B = 16
_S = 16
_P = _S * _S
_PAD = 128

_TAPS = [(k, 16 * (k // 3 - 1) + (k % 3 - 1), k % 3 - 1) for k in range(9)]


def _gate_kernel(meta_ref, w1_ref, b1_ref, w2_ref, b2_ref, top1_ref, bal_ref):
    meta = meta_ref[:]
    h = jax.lax.dot_general(meta, w1_ref[:], (((1,), (1,)), ((), ())),
                            preferred_element_type=jnp.float32)
    h = jnp.maximum(h + b1_ref[:], 0.0)
    logits = jax.lax.dot_general(h, w2_ref[:], (((1,), (1,)), ((), ())),
                                 preferred_element_type=jnp.float32)
    logits = logits + b2_ref[:]
    mx = jnp.max(logits, axis=1, keepdims=True)
    ex = jnp.exp(logits - mx)
    probs = ex / jnp.sum(ex, axis=1, keepdims=True)
    lane = jax.lax.broadcasted_iota(jnp.int32, (_B, _E), 1)
    is_max = logits == mx
    top1 = jnp.min(jnp.where(is_max, lane, _E), axis=1, keepdims=True)
    top1_ref[:] = top1
    imp = jnp.sum(probs, axis=0, keepdims=True)
    imp = imp / (jnp.sum(imp, axis=1, keepdims=True) + 1e-8)
    mean = jnp.sum(imp, axis=1, keepdims=True) / _E
    var = jnp.sum((imp - mean) ** 2, axis=1, keepdims=True) / (_E - 1)
    bal_ref[:, :] = jnp.sqrt(var)


def _moe_kernel(top1_ref, x_ref, w1_ref, w2_ref,
                bn1_g_ref, bn1_b_ref, bn2_g_ref, bn2_b_ref,
                out_ref, h1_ref, h2_ref, xpad_ref, hpad_ref,
                sc1_ref, sh1_ref, sc2_ref, sh2_ref, wbuf_ref, wbf_ref, sem):
    f32 = jnp.float32
    bf16 = jnp.bfloat16
    row = jax.lax.broadcasted_iota(jnp.int32, (_P, 1), 0)
    mask_m = (row % _S != 0).astype(f32)
    mask_p = (row % _S != _S - 1).astype(f32)

    def fetch(w_ref, b, slot):
        e = top1_ref[b, 0]
        pltpu.make_async_copy(
            w_ref.at[e], wbuf_ref.at[slot], sem.at[slot]).start()

    def wait(w_ref, b, slot):
        e = top1_ref[b, 0]
        pltpu.make_async_copy(
            w_ref.at[e], wbuf_ref.at[slot], sem.at[slot]).wait()

    def conv(src, slot):
        by_dx = {-1: None, 0: None, 1: None}
        for k, off, dx in _TAPS:
            xs = src[_PAD + off:_PAD + off + _P, :]
            t = jax.lax.dot_general(xs, wbf_ref[slot, k],
                                    (((1,), (1,)), ((), ())),
                                    preferred_element_type=f32)
            by_dx[dx] = t if by_dx[dx] is None else by_dx[dx] + t
        return by_dx[-1] * mask_m + by_dx[0] + by_dx[1] * mask_p

    def affine(ssum, ssq, cnt, g_ref, b_ref, sc_ref, sh_ref):
        for e in range(_E):
            n = jnp.maximum(cnt[e:e + 1, :], 1.0) * _P
            m = ssum[e:e + 1, :] / n
            v = ssq[e:e + 1, :] / n - m * m
            sc = g_ref[e:e + 1, :] * jax.lax.rsqrt(v + 1e-5)
            sc_ref[e] = sc
            sh_ref[e] = b_ref[e:e + 1, :] - m * sc

    xpad_ref[:, :, :] = jnp.zeros((_B, _P + 2 * _PAD, _C), bf16)
    hpad_ref[:, :, :] = jnp.zeros((2, _P + 2 * _PAD, _C), bf16)

    def stage(b, _):
        xpad_ref[b, _PAD:_PAD + _P, :] = x_ref[b].astype(bf16)
        return 0

    jax.lax.fori_loop(0, _B, stage, 0)

    zstat = jnp.zeros((_E, _C), f32)
    zcnt = jnp.zeros((_E, 1), f32)
    fetch(w1_ref, 0, 0)
    fetch(w1_ref, 1, 1)

    def stats(h, e, ssum, ssq, cnt):
        oh = (jax.lax.broadcasted_iota(jnp.int32, (_E, 1), 0) == e).astype(f32)
        return (ssum + oh * jnp.sum(h, axis=0, keepdims=True),
                ssq + oh * jnp.sum(h * h, axis=0, keepdims=True),
                cnt + oh)

    def pass1(i, carry):
        ssum, ssq, cnt = carry
        b0 = 2 * i
        b1 = b0 + 1
        s0 = jnp.bitwise_and(b0, 3)
        s1 = jnp.bitwise_and(b1, 3)
        jax.lax.cond(b0 + 2 < _B,
                     lambda: fetch(w1_ref, b0 + 2, jnp.bitwise_and(b0 + 2, 3)),
                     lambda: None)
        jax.lax.cond(b1 + 2 < _B,
                     lambda: fetch(w1_ref, b1 + 2, jnp.bitwise_and(b1 + 2, 3)),
                     lambda: None)
        wait(w1_ref, b0, s0)
        wbf_ref[s0] = wbuf_ref[s0].astype(bf16)
        wait(w1_ref, b1, s1)
        wbf_ref[s1] = wbuf_ref[s1].astype(bf16)
        h0 = conv(xpad_ref.at[b0], s0)
        h1 = conv(xpad_ref.at[b1], s1)
        h1_ref[b0] = h0
        h1_ref[b1] = h1
        ssum, ssq, cnt = stats(h0, top1_ref[b0, 0], ssum, ssq, cnt)
        ssum, ssq, cnt = stats(h1, top1_ref[b1, 0], ssum, ssq, cnt)
        return ssum, ssq, cnt

    ssum1, ssq1, cnt = jax.lax.fori_loop(0, _B // 2, pass1,
                                         (zstat, zstat, zcnt))
    affine(ssum1, ssq1, cnt, bn1_g_ref, bn1_b_ref, sc1_ref, sh1_ref)

    fetch(w2_ref, 0, 0)
    fetch(w2_ref, 1, 1)

    def pass2(i, carry):
        ssum, ssq, cnt2 = carry
        b0 = 2 * i
        b1 = b0 + 1
        e0 = top1_ref[b0, 0]
        e1 = top1_ref[b1, 0]
        s0 = jnp.bitwise_and(b0, 3)
        s1 = jnp.bitwise_and(b1, 3)
        jax.lax.cond(b0 + 2 < _B,
                     lambda: fetch(w2_ref, b0 + 2, jnp.bitwise_and(b0 + 2, 3)),
                     lambda: None)
        jax.lax.cond(b1 + 2 < _B,
                     lambda: fetch(w2_ref, b1 + 2, jnp.bitwise_and(b1 + 2, 3)),
                     lambda: None)
        hn0 = jnp.maximum(h1_ref[b0] * sc1_ref[e0] + sh1_ref[e0], 0.0)
        hpad_ref[0, _PAD:_PAD + _P, :] = hn0.astype(bf16)
        hn1 = jnp.maximum(h1_ref[b1] * sc1_ref[e1] + sh1_ref[e1], 0.0)
        hpad_ref[1, _PAD:_PAD + _P, :] = hn1.astype(bf16)
        wait(w2_ref, b0, s0)
        wbf_ref[s0] = wbuf_ref[s0].astype(bf16)
        wait(w2_ref, b1, s1)
        wbf_ref[s1] = wbuf_ref[s1].astype(bf16)
        h0 = conv(hpad_ref.at[0], s0)
        h1 = conv(hpad_ref.at[1], s1)
        h2_ref[b0] = h0
        h2_ref[b1] = h1
        ssum, ssq, cnt2 = stats(h0, e0, ssum, ssq, cnt2)
        ssum, ssq, cnt2 = stats(h1, e1, ssum, ssq, cnt2)
        return ssum, ssq, cnt2

    ssum2, ssq2, _ = jax.lax.fori_loop(0, _B // 2, pass2,
                                       (zstat, zstat, zcnt))
    affine(ssum2, ssq2, cnt, bn2_g_ref, bn2_b_ref, sc2_ref, sh2_ref)

    def pass3(b, _):
        e = top1_ref[b, 0]
        out_ref[b] = jnp.maximum(h2_ref[b] * sc2_ref[e] + sh2_ref[e], 0.0)
        return 0

    jax.lax.fori_loop(0, _B, pass3, 0)


@functools.partial(jax.jit, static_argnames=("interpret",))
def kernel(moe_c4, meta, gate_w1, gate_b1, gate_w2, gate_b2, conv1_w, conv1_b,
           bn1_g, bn1_b, conv2_w, conv2_b, bn2_g, bn2_b, interpret=False):
    del conv1_b, conv2_b
    f32 = jnp.float32

    top1, bal = pl.pallas_call(
        _gate_kernel,
        out_shape=(jax.ShapeDtypeStruct((_B, 1), jnp.int32),
                   jax.ShapeDtypeStruct((1, 1), f32)),
        interpret=interpret,
    )(meta, gate_w1, gate_b1[None, :], gate_w2, gate_b2[None, :])

    x_nat = moe_c4.transpose(0, 2, 3, 1).reshape(_B, _P, _C)
    w1p = conv1_w.transpose(0, 3, 4, 1, 2).reshape(_E, 9, _HID, _C)
    w2p = conv2_w.transpose(0, 3, 4, 1, 2).reshape(_E, 9, _C, _HID)

    out = pl.pallas_call(
        _moe_kernel,
        out_shape=jax.ShapeDtypeStruct((_B, _P, _C), f32),
        in_specs=[
            pl.BlockSpec(memory_space=pltpu.SMEM),
            pl.BlockSpec(memory_space=pltpu.VMEM),
            pl.BlockSpec(memory_space=pltpu.MemorySpace.HBM),
            pl.BlockSpec(memory_space=pltpu.MemorySpace.HBM),
            pl.BlockSpec(memory_space=pltpu.VMEM),
            pl.BlockSpec(memory_space=pltpu.VMEM),
            pl.BlockSpec(memory_space=pltpu.VMEM),
            pl.BlockSpec(memory_space=pltpu.VMEM),
        ],
        scratch_shapes=[
            pltpu.VMEM((_B, _P, _HID), f32),
            pltpu.VMEM((_B, _P, _C), f32),
            pltpu.VMEM((_B, _P + 2 * _PAD, _C), jnp.bfloat16),
            pltpu.VMEM((2, _P + 2 * _PAD, _C), jnp.bfloat16),
            pltpu.VMEM((_E, 1, _HID), f32),
            pltpu.VMEM((_E, 1, _HID), f32),
            pltpu.VMEM((_E, 1, _C), f32),
            pltpu.VMEM((_E, 1, _C), f32),
            pltpu.VMEM((4, 9, _HID, _C), f32),
            pltpu.VMEM((4, 9, _HID, _C), jnp.bfloat16),
            pltpu.SemaphoreType.DMA((4,)),
        ],
        interpret=interpret,
    )(top1, x_nat, w1p, w2p, bn1_g, bn1_b, bn2_g, bn2_b)

    return out.reshape(_B, _S, _S, _C).transpose(0, 3, 1, 2), bal[0, 0]


# speedup vs baseline: 1.3564x; 1.0776x over previous
import functools

import jax
import jax.numpy as jnp
from jax.experimental import pallas as pl
from jax.experimental.pallas import tpu as pltpu

_E = 8
_C = 192
_HID = 192
_B = 16
_S = 16
_P = _S * _S
_PAD = 128

_TAPS = [(k, 16 * (k // 3 - 1) + (k % 3 - 1), k % 3 - 1) for k in range(9)]


def _gate_kernel(meta_ref, w1_ref, b1_ref, w2_ref, b2_ref, top1_ref, bal_ref):
    meta = meta_ref[:]
    h = jax.lax.dot_general(meta, w1_ref[:], (((1,), (1,)), ((), ())),
                            preferred_element_type=jnp.float32)
    h = jnp.maximum(h + b1_ref[:], 0.0)
    logits = jax.lax.dot_general(h, w2_ref[:], (((1,), (1,)), ((), ())),
                                 preferred_element_type=jnp.float32)
    logits = logits + b2_ref[:]
    mx = jnp.max(logits, axis=1, keepdims=True)
    ex = jnp.exp(logits - mx)
    probs = ex / jnp.sum(ex, axis=1, keepdims=True)
    lane = jax.lax.broadcasted_iota(jnp.int32, (_B, _E), 1)
    is_max = logits == mx
    top1 = jnp.min(jnp.where(is_max, lane, _E), axis=1, keepdims=True)
    top1_ref[:] = top1
    imp = jnp.sum(probs, axis=0, keepdims=True)
    imp = imp / (jnp.sum(imp, axis=1, keepdims=True) + 1e-8)
    mean = jnp.sum(imp, axis=1, keepdims=True) / _E
    var = jnp.sum((imp - mean) ** 2, axis=1, keepdims=True) / (_E - 1)
    bal_ref[:, :] = jnp.sqrt(var)


def _moe_kernel(top1_ref, x_ref, w1_ref, w2_ref,
                bn1_g_ref, bn1_b_ref, bn2_g_ref, bn2_b_ref,
                out_ref, h1_ref, h2_ref, xpad_ref, hpad_ref,
                sc1_ref, sh1_ref, sc2_ref, sh2_ref, wbuf_ref, wbf_ref, sem):
    f32 = jnp.float32
    bf16 = jnp.bfloat16
    row = jax.lax.broadcasted_iota(jnp.int32, (_P, 1), 0)
    mask_m = (row % _S != 0).astype(f32)
    mask_p = (row % _S != _S - 1).astype(f32)

    def fetch(w_ref, b, slot):
        e = top1_ref[b, 0]
        pltpu.make_async_copy(
            w_ref.at[e], wbuf_ref.at[slot], sem.at[slot]).start()

    def wait(w_ref, b, slot):
        e = top1_ref[b, 0]
        pltpu.make_async_copy(
            w_ref.at[e], wbuf_ref.at[slot], sem.at[slot]).wait()

    def conv(src, slot):
        by_dx = {-1: None, 0: None, 1: None}
        for k, off, dx in _TAPS:
            xs = src[_PAD + off:_PAD + off + _P, :]
            t = jax.lax.dot_general(xs, wbf_ref[slot, k],
                                    (((1,), (1,)), ((), ())),
                                    preferred_element_type=f32)
            by_dx[dx] = t if by_dx[dx] is None else by_dx[dx] + t
        return by_dx[-1] * mask_m + by_dx[0] + by_dx[1] * mask_p

    def affine(ssum, ssq, cnt, g_ref, b_ref, sc_ref, sh_ref):
        for e in range(_E):
            n = jnp.maximum(cnt[e:e + 1, :], 1.0) * _P
            m = ssum[e:e + 1, :] / n
            v = ssq[e:e + 1, :] / n - m * m
            sc = g_ref[e:e + 1, :] * jax.lax.rsqrt(v + 1e-5)
            sc_ref[e] = sc
            sh_ref[e] = b_ref[e:e + 1, :] - m * sc

    xpad_ref[:, :, :] = jnp.zeros((_B, _P + 2 * _PAD, _C), bf16)
    hpad_ref[:, :, :] = jnp.zeros((4, _P + 2 * _PAD, _C), bf16)

    def stage(b, _):
        xpad_ref[b, _PAD:_PAD + _P, :] = x_ref[b].astype(bf16)
        return 0

    jax.lax.fori_loop(0, _B, stage, 0)

    zstat = jnp.zeros((_E, _C), f32)
    zcnt = jnp.zeros((_E, 1), f32)
    for j in range(4):
        fetch(w1_ref, j, j)

    def stats(h, e, ssum, ssq, cnt):
        oh = (jax.lax.broadcasted_iota(jnp.int32, (_E, 1), 0) == e).astype(f32)
        return (ssum + oh * jnp.sum(h, axis=0, keepdims=True),
                ssq + oh * jnp.sum(h * h, axis=0, keepdims=True),
                cnt + oh)

    def pass1(i, carry):
        ssum, ssq, cnt = carry
        bs = [4 * i + j for j in range(4)]
        ss = [jnp.bitwise_and(b, 7) for b in bs]
        for b in bs:
            jax.lax.cond(b + 4 < _B,
                         lambda b=b: fetch(w1_ref, b + 4,
                                           jnp.bitwise_and(b + 4, 7)),
                         lambda: None)
        for b, s in zip(bs, ss):
            wait(w1_ref, b, s)
            wbf_ref[s] = wbuf_ref[s].astype(bf16)
        hs = [conv(xpad_ref.at[b], s) for b, s in zip(bs, ss)]
        for b, h in zip(bs, hs):
            h1_ref[b] = h
        for b, h in zip(bs, hs):
            ssum, ssq, cnt = stats(h, top1_ref[b, 0], ssum, ssq, cnt)
        return ssum, ssq, cnt

    ssum1, ssq1, cnt = jax.lax.fori_loop(0, _B // 4, pass1,
                                         (zstat, zstat, zcnt))
    affine(ssum1, ssq1, cnt, bn1_g_ref, bn1_b_ref, sc1_ref, sh1_ref)

    for j in range(4):
        fetch(w2_ref, j, j)

    def pass2(i, carry):
        ssum, ssq, cnt2 = carry
        bs = [4 * i + j for j in range(4)]
        es = [top1_ref[b, 0] for b in bs]
        ss = [jnp.bitwise_and(b, 7) for b in bs]
        for b in bs:
            jax.lax.cond(b + 4 < _B,
                         lambda b=b: fetch(w2_ref, b + 4,
                                           jnp.bitwise_and(b + 4, 7)),
                         lambda: None)
        for j, (b, e) in enumerate(zip(bs, es)):
            hn = jnp.maximum(h1_ref[b] * sc1_ref[e] + sh1_ref[e], 0.0)
            hpad_ref[j, _PAD:_PAD + _P, :] = hn.astype(bf16)
        for b, s in zip(bs, ss):
            wait(w2_ref, b, s)
            wbf_ref[s] = wbuf_ref[s].astype(bf16)
        hs = [conv(hpad_ref.at[j], s) for j, s in enumerate(ss)]
        for b, h in zip(bs, hs):
            h2_ref[b] = h
        for e, h in zip(es, hs):
            ssum, ssq, cnt2 = stats(h, e, ssum, ssq, cnt2)
        return ssum, ssq, cnt2

    ssum2, ssq2, _ = jax.lax.fori_loop(0, _B // 4, pass2,
                                       (zstat, zstat, zcnt))
    affine(ssum2, ssq2, cnt, bn2_g_ref, bn2_b_ref, sc2_ref, sh2_ref)

    def pass3(b, _):
        e = top1_ref[b, 0]
        out_ref[b] = jnp.maximum(h2_ref[b] * sc2_ref[e] + sh2_ref[e], 0.0)
        return 0

    jax.lax.fori_loop(0, _B, pass3, 0)


@functools.partial(jax.jit, static_argnames=("interpret",))
def kernel(moe_c4, meta, gate_w1, gate_b1, gate_w2, gate_b2, conv1_w, conv1_b,
           bn1_g, bn1_b, conv2_w, conv2_b, bn2_g, bn2_b, interpret=False):
    del conv1_b, conv2_b
    f32 = jnp.float32

    top1, bal = pl.pallas_call(
        _gate_kernel,
        out_shape=(jax.ShapeDtypeStruct((_B, 1), jnp.int32),
                   jax.ShapeDtypeStruct((1, 1), f32)),
        interpret=interpret,
    )(meta, gate_w1, gate_b1[None, :], gate_w2, gate_b2[None, :])

    x_nat = moe_c4.transpose(0, 2, 3, 1).reshape(_B, _P, _C)
    w1p = conv1_w.transpose(0, 3, 4, 1, 2).reshape(_E, 9, _HID, _C)
    w2p = conv2_w.transpose(0, 3, 4, 1, 2).reshape(_E, 9, _C, _HID)

    out = pl.pallas_call(
        _moe_kernel,
        out_shape=jax.ShapeDtypeStruct((_B, _P, _C), f32),
        in_specs=[
            pl.BlockSpec(memory_space=pltpu.SMEM),
            pl.BlockSpec(memory_space=pltpu.VMEM),
            pl.BlockSpec(memory_space=pltpu.MemorySpace.HBM),
            pl.BlockSpec(memory_space=pltpu.MemorySpace.HBM),
            pl.BlockSpec(memory_space=pltpu.VMEM),
            pl.BlockSpec(memory_space=pltpu.VMEM),
            pl.BlockSpec(memory_space=pltpu.VMEM),
            pl.BlockSpec(memory_space=pltpu.VMEM),
        ],
        scratch_shapes=[
            pltpu.VMEM((_B, _P, _HID), f32),
            pltpu.VMEM((_B, _P, _C), f32),
            pltpu.VMEM((_B, _P + 2 * _PAD, _C), jnp.bfloat16),
            pltpu.VMEM((4, _P + 2 * _PAD, _C), jnp.bfloat16),
            pltpu.VMEM((_E, 1, _HID), f32),
            pltpu.VMEM((_E, 1, _HID), f32),
            pltpu.VMEM((_E, 1, _C), f32),
            pltpu.VMEM((_E, 1, _C), f32),
            pltpu.VMEM((8, 9, _HID, _C), f32),
            pltpu.VMEM((8, 9, _HID, _C), jnp.bfloat16),
            pltpu.SemaphoreType.DMA((8,)),
        ],
        interpret=interpret,
    )(top1, x_nat, w1p, w2p, bn1_g, bn1_b, bn2_g, bn2_b)

    return out.reshape(_B, _S, _S, _C).transpose(0, 3, 1, 2), bal[0, 0]
